# baseline (speedup 1.0000x reference)
import functools

import jax
import jax.numpy as jnp
from jax import lax
from jax.experimental import pallas as pl
from jax.experimental.pallas import tpu as pltpu
from jax.experimental.pallas import tpu_sc as plsc

N = 10000
E = 320000
D_IN = 128
D_HID = 16
D_OUT = 7

NC = 2
NS = 16
NW = NC * NS
EB = 128
NP = 10240
RPS = NP // NS

EPW = 10240
EPAD = NW * EPW
C = 4 * EB
NCHUNK = EPW // C
SINK = NP - 1


def _sc_mesh():
    return plsc.VectorSubcoreMesh(
        core_axis_name="c", subcore_axis_name="s",
        num_cores=NC, num_subcores=NS)


def _deg_scatter(edge, ones_rows, zeros_rows):
    @functools.partial(
        pl.kernel,
        out_type=jax.ShapeDtypeStruct((NC * NP, 16), jnp.float32),
        mesh=_sc_mesh(),
        scratch_types=[
            pltpu.VMEM((3, C), jnp.int32),
            pltpu.VMEM((EB, 16), jnp.float32),
            pltpu.VMEM_SHARED((NP, 16), jnp.float32),
            pltpu.SemaphoreType.DMA((3,)),
            pltpu.SemaphoreType.DMA((4,)),
        ],
        compiler_params=pltpu.CompilerParams(use_tc_tiling_on_sc=False),
    )
    def k(edge_h, ones_h, zeros_h, out_h, didx, ones_v, acc, sem_i, sem_sc):
        cid = lax.axis_index("c")
        sid = lax.axis_index("s")
        wid = sid * NC + cid
        woff = wid * EPW
        pltpu.sync_copy(ones_h, ones_v)
        pltpu.sync_copy(zeros_h, acc.at[pl.ds(sid * RPS, RPS)])
        plsc.subcore_barrier()

        def idx_copy(i, slot):
            return pltpu.make_async_copy(
                edge_h.at[1, pl.ds(woff + i * C, C)], didx.at[slot],
                sem_i.at[slot])

        def sc_copy(slot, b):
            return pltpu.make_async_copy(
                ones_v, acc.at[didx.at[slot, pl.ds(b * EB, EB)]],
                sem_sc.at[b])

        idx_copy(0, 0).start()
        idx_copy(1, 1).start()

        def body(i, carry):
            slot = lax.rem(i, 3)
            idx_copy(i, slot).wait()
            for b in range(4):
                @pl.when(i > 0)
                def _():
                    sc_copy(slot, b).wait()
                pltpu.async_copy(
                    ones_v, acc.at[didx.at[slot, pl.ds(b * EB, EB)]],
                    sem_sc.at[b], add=True)

            @pl.when(i < NCHUNK - 2)
            def _():
                idx_copy(i + 2, lax.rem(i + 2, 3)).start()
            return carry

        lax.fori_loop(0, NCHUNK, body, 0)
        for b in range(4):
            sc_copy(0, b).wait()
        plsc.subcore_barrier()
        pltpu.sync_copy(acc.at[pl.ds(sid * RPS, RPS)],
                        out_h.at[pl.ds(cid * NP + sid * RPS, RPS)])

    return k(edge, ones_rows, zeros_rows)


def _edge_scatter(table, edge, zeros_rows):
    @functools.partial(
        pl.kernel,
        out_type=jax.ShapeDtypeStruct((NC * NP, 16), jnp.float32),
        mesh=_sc_mesh(),
        scratch_types=[
            pltpu.VMEM((3, C), jnp.int32),
            pltpu.VMEM((3, C), jnp.int32),
            pltpu.VMEM((4, EB, 16), jnp.float32),
            pltpu.VMEM_SHARED((NP, 16), jnp.float32),
            pltpu.VMEM_SHARED((NP, 16), jnp.float32),
            pltpu.SemaphoreType.DMA((3,)),
            pltpu.SemaphoreType.DMA((4,)),
            pltpu.SemaphoreType.DMA((4,)),
        ],
        compiler_params=pltpu.CompilerParams(use_tc_tiling_on_sc=False),
    )
    def k(table_h, edge_h, zeros_h, out_h,
          sidx, didx, rows, acc, tbl, sem_i, sem_g, sem_sc):
        cid = lax.axis_index("c")
        sid = lax.axis_index("s")
        wid = sid * NC + cid
        woff = wid * EPW
        pltpu.sync_copy(zeros_h, acc.at[pl.ds(sid * RPS, RPS)])
        @pl.when(sid < 15)
        def _():
            pltpu.sync_copy(table_h.at[pl.ds(sid * 624, 624)],
                            tbl.at[pl.ds(sid * 624, 624)])

        @pl.when(sid == 15)
        def _():
            pltpu.sync_copy(table_h.at[pl.ds(9360, 640)],
                            tbl.at[pl.ds(9360, 640)])

        @pl.when(sid == 0)
        def _():
            pltpu.sync_copy(zeros_h.at[pl.ds(0, NP - N)],
                            tbl.at[pl.ds(N, NP - N)])

        plsc.subcore_barrier()

        def sidx_copy(i, slot):
            return pltpu.make_async_copy(
                edge_h.at[0, pl.ds(woff + i * C, C)], sidx.at[slot],
                sem_i.at[slot])

        def didx_copy(i, slot):
            return pltpu.make_async_copy(
                edge_h.at[1, pl.ds(woff + i * C, C)], didx.at[slot],
                sem_i.at[slot])

        def gather_copy(slot, b):
            return pltpu.make_async_copy(
                tbl.at[sidx.at[slot, pl.ds(b * EB, EB)]], rows.at[b],
                sem_g.at[b])

        def sc_copy(slot, b):
            return pltpu.make_async_copy(
                rows.at[b], acc.at[didx.at[slot, pl.ds(b * EB, EB)]],
                sem_sc.at[b])

        sidx_copy(0, 0).start()
        didx_copy(0, 0).start()
        sidx_copy(1, 1).start()
        didx_copy(1, 1).start()

        def body(i, carry):
            slot = lax.rem(i, 3)
            sidx_copy(i, slot).wait()
            didx_copy(i, slot).wait()
            for b in range(4):
                @pl.when(i > 0)
                def _():
                    sc_copy(slot, b).wait()
                gather_copy(slot, b).start()
            for b in range(4):
                gather_copy(slot, b).wait()
                pltpu.async_copy(
                    rows.at[b], acc.at[didx.at[slot, pl.ds(b * EB, EB)]],
                    sem_sc.at[b], add=True)

            @pl.when(i < NCHUNK - 2)
            def _():
                sidx_copy(i + 2, lax.rem(i + 2, 3)).start()
                didx_copy(i + 2, lax.rem(i + 2, 3)).start()
            return carry

        lax.fori_loop(0, NCHUNK, body, 0)
        for b in range(4):
            sc_copy(0, b).wait()
        plsc.subcore_barrier()
        pltpu.sync_copy(acc.at[pl.ds(sid * RPS, RPS)],
                        out_h.at[pl.ds(cid * NP + sid * RPS, RPS)])

    return k(table, edge, zeros_rows)


NPK = N // 8
NPPK = NP // 8


def _tc_xw1(x, w1):
    def body(x_ref, w_ref, o_ref):
        o_ref[...] = jnp.dot(x_ref[...], w_ref[...],
                             preferred_element_type=jnp.float32)

    return pl.pallas_call(
        body,
        out_shape=jax.ShapeDtypeStruct((N, 16), jnp.float32),
    )(x, w1)


def _tc_scale(degp, xw1):
    def body(degp_ref, xw_ref, g_ref, dinv_ref):
        deg = degp_ref[pl.ds(0, NPK)] + degp_ref[pl.ds(NPPK, NPK)] + 1.0
        dinv = lax.rsqrt(deg)
        g_ref[...] = xw_ref[...] * dinv
        dinv_ref[...] = dinv

    return pl.pallas_call(
        body,
        out_shape=(jax.ShapeDtypeStruct((NPK, 128), jnp.float32),
                   jax.ShapeDtypeStruct((NPK, 128), jnp.float32)),
    )(degp, xw1)


def _tc_mid(accp, g1, dinv, w2bd, b1p):
    def body(accp_ref, g_ref, dinv_ref, w_ref, b_ref, g2_ref):
        z = dinv_ref[...] * (
            accp_ref[pl.ds(0, NPK)] + accp_ref[pl.ds(NPPK, NPK)] + g_ref[...])
        z = z + b_ref[...]
        h = jnp.maximum(z, 0.0)
        h2 = jnp.dot(h, w_ref[...], preferred_element_type=jnp.float32)
        g2_ref[...] = h2 * dinv_ref[...]

    return pl.pallas_call(
        body,
        out_shape=jax.ShapeDtypeStruct((NPK, 128), jnp.float32),
    )(accp, g1, dinv, w2bd, b1p)


def _tc_zfinal(accp, g2, dinv, b2p):
    def body(accp_ref, g_ref, dinv_ref, b_ref, o_ref):
        zp = dinv_ref[...] * (
            accp_ref[pl.ds(0, NPK)] + accp_ref[pl.ds(NPPK, NPK)] + g_ref[...])
        o_ref[...] = zp + b_ref[...]

    return pl.pallas_call(
        body,
        out_shape=jax.ShapeDtypeStruct((NPK, 128), jnp.float32),
    )(accp, g2, dinv, b2p)


def _tc_softmax(z):
    def body(z_ref, o_ref):
        z = z_ref[...]
        col = lax.broadcasted_iota(jnp.int32, (N, 16), 1)
        zm = jnp.where(col < D_OUT, z, -jnp.inf)
        m = jnp.max(zm, axis=1, keepdims=True)
        e = jnp.exp(zm - m)
        lse = jnp.log(jnp.sum(e, axis=1, keepdims=True)) + m
        o_ref[...] = (z - lse)[:, :D_OUT]

    return pl.pallas_call(
        body,
        out_shape=jax.ShapeDtypeStruct((N, D_OUT), jnp.float32),
    )(z)


@jax.jit
def kernel(x, edge_index, W1, b1, W2, b2):
    edge = jnp.pad(edge_index.astype(jnp.int32),
                   ((0, 0), (0, EPAD - E)), constant_values=SINK)
    zeros_rows = jnp.zeros((RPS, 16), jnp.float32)
    ones_rows = jnp.ones((EB, 16), jnp.float32)

    xw1 = _tc_xw1(x, W1).reshape(NPK, 128)
    degp = _deg_scatter(edge, ones_rows, zeros_rows)
    g1, dinv = _tc_scale(degp.reshape(2 * NPPK, 128), xw1)

    acc1 = _edge_scatter(g1.reshape(N, 16), edge, zeros_rows)

    w2p = jnp.pad(W2, ((0, 0), (0, 16 - D_OUT)))
    w2bd = jnp.kron(jnp.eye(8, dtype=jnp.float32), w2p)
    b1p = jnp.tile(b1, 8).reshape(1, 128)
    b2p = jnp.tile(jnp.pad(b2, (0, 16 - D_OUT)), 8).reshape(1, 128)

    g2 = _tc_mid(acc1.reshape(2 * NPPK, 128), g1, dinv, w2bd, b1p)
    acc2 = _edge_scatter(g2.reshape(N, 16), edge, zeros_rows)
    z2 = _tc_zfinal(acc2.reshape(2 * NPPK, 128), g2, dinv, b2p)
    return _tc_softmax(z2.reshape(N, 16))


# speedup vs baseline: 1.0061x; 1.0061x over previous
import functools

import jax
import jax.numpy as jnp
from jax import lax
from jax.experimental import pallas as pl
from jax.experimental.pallas import tpu as pltpu
from jax.experimental.pallas import tpu_sc as plsc

N = 10000
E = 320000
D_IN = 128
D_HID = 16
D_OUT = 7

NC = 2
NS = 16
NW = NC * NS
EB = 128
NBATCH = E // EB
BPT = -(-NBATCH // NW)
NP = 10240
RPS = NP // NS


def _sc_mesh():
    return plsc.VectorSubcoreMesh(
        core_axis_name="c", subcore_axis_name="s",
        num_cores=NC, num_subcores=NS)


def _deg_scatter(edge, ones_rows, zeros_rows):
    @functools.partial(
        pl.kernel,
        out_type=jax.ShapeDtypeStruct((NC * NP, 16), jnp.float32),
        mesh=_sc_mesh(),
        scratch_types=[
            pltpu.VMEM((4, EB), jnp.int32),
            pltpu.VMEM((EB, 16), jnp.float32),
            pltpu.VMEM_SHARED((NP, 16), jnp.float32),
            pltpu.SemaphoreType.DMA((4,)),
            pltpu.SemaphoreType.DMA((4,)),
        ],
        compiler_params=pltpu.CompilerParams(use_tc_tiling_on_sc=False),
    )
    def k(edge_h, ones_h, zeros_h, out_h, didx, ones_v, acc, sem_i, sem_sc):
        cid = lax.axis_index("c")
        sid = lax.axis_index("s")
        wid = sid * NC + cid
        pltpu.sync_copy(ones_h, ones_v)
        pltpu.sync_copy(zeros_h, acc.at[pl.ds(sid * RPS, RPS)])
        plsc.subcore_barrier()

        def valid(t):
            return jnp.logical_and(t >= 0, (wid + t * NW) < NBATCH)

        def boff(t):
            return (wid + t * NW) * EB

        def idx_copy(t, s):
            return pltpu.make_async_copy(
                edge_h.at[1, pl.ds(boff(t), EB)], didx.at[s], sem_i.at[s])

        def sc_desc(s):
            return pltpu.make_async_copy(
                ones_v, acc.at[didx.at[s]], sem_sc.at[s])

        def issue_idx(t, s):
            @pl.when(valid(t))
            def _():
                idx_copy(t, s).start()

        def wait_sc(t, s):
            @pl.when(valid(t))
            def _():
                sc_desc(s).wait()

        def scatter(t, s):
            @pl.when(valid(t))
            def _():
                idx_copy(t, s).wait()
                pltpu.async_copy(ones_v, acc.at[didx.at[s]], sem_sc.at[s],
                                 add=True)

        issue_idx(0, 0)
        issue_idx(1, 1)

        def body(T4, carry):
            T = T4 * 4
            for s in range(4):
                t = T + s
                wait_sc(t - 2, (s + 2) % 4)
                issue_idx(t + 2, (s + 2) % 4)
                scatter(t, s)
            return carry

        lax.fori_loop(0, BPT // 4 + 1, body, 0)
        wait_sc(BPT - 1, (BPT - 1) % 4)
        wait_sc(BPT, BPT % 4)
        plsc.subcore_barrier()
        pltpu.sync_copy(acc.at[pl.ds(sid * RPS, RPS)],
                        out_h.at[pl.ds(cid * NP + sid * RPS, RPS)])

    return k(edge, ones_rows, zeros_rows)


def _edge_scatter(table, edge, zeros_rows):
    @functools.partial(
        pl.kernel,
        out_type=jax.ShapeDtypeStruct((NC * NP, 16), jnp.float32),
        mesh=_sc_mesh(),
        scratch_types=[
            pltpu.VMEM((4, EB), jnp.int32),
            pltpu.VMEM((4, EB), jnp.int32),
            pltpu.VMEM((4, EB, 16), jnp.float32),
            pltpu.VMEM_SHARED((NP, 16), jnp.float32),
            pltpu.VMEM_SHARED((N, 16), jnp.float32),
            pltpu.SemaphoreType.DMA((4,)),
            pltpu.SemaphoreType.DMA((4,)),
            pltpu.SemaphoreType.DMA((4,)),
        ],
        compiler_params=pltpu.CompilerParams(use_tc_tiling_on_sc=False),
    )
    def k(table_h, edge_h, zeros_h, out_h,
          sidx, didx, rows, acc, tbl, sem_i, sem_g, sem_sc):
        cid = lax.axis_index("c")
        sid = lax.axis_index("s")
        wid = sid * NC + cid
        pltpu.sync_copy(zeros_h, acc.at[pl.ds(sid * RPS, RPS)])
        @pl.when(sid < 15)
        def _():
            pltpu.sync_copy(table_h.at[pl.ds(sid * 624, 624)],
                            tbl.at[pl.ds(sid * 624, 624)])

        @pl.when(sid == 15)
        def _():
            pltpu.sync_copy(table_h.at[pl.ds(9360, 640)],
                            tbl.at[pl.ds(9360, 640)])

        plsc.subcore_barrier()

        def valid(t):
            return jnp.logical_and(t >= 0, (wid + t * NW) < NBATCH)

        def boff(t):
            return (wid + t * NW) * EB

        def sidx_copy(t, s):
            return pltpu.make_async_copy(
                edge_h.at[0, pl.ds(boff(t), EB)], sidx.at[s], sem_i.at[s])

        def didx_copy(t, s):
            return pltpu.make_async_copy(
                edge_h.at[1, pl.ds(boff(t), EB)], didx.at[s], sem_i.at[s])

        def gather_copy(s):
            return pltpu.make_async_copy(
                tbl.at[sidx.at[s]], rows.at[s], sem_g.at[s])

        def sc_desc(s):
            return pltpu.make_async_copy(
                rows.at[s], acc.at[didx.at[s]], sem_sc.at[s])

        def issue_idx(t, s):
            @pl.when(valid(t))
            def _():
                sidx_copy(t, s).start()
                didx_copy(t, s).start()

        def wait_sc(t, s):
            @pl.when(valid(t))
            def _():
                sc_desc(s).wait()

        def start_gather(t, s):
            @pl.when(valid(t))
            def _():
                sidx_copy(t, s).wait()
                didx_copy(t, s).wait()
                gather_copy(s).start()

        def scatter(t, s):
            @pl.when(valid(t))
            def _():
                gather_copy(s).wait()
                pltpu.async_copy(rows.at[s], acc.at[didx.at[s]],
                                 sem_sc.at[s], add=True)

        issue_idx(0, 0)
        issue_idx(1, 1)

        def body(T4, carry):
            T = T4 * 4
            for s in range(4):
                t = T + s
                wait_sc(t - 2, (s + 2) % 4)
                issue_idx(t + 2, (s + 2) % 4)
                start_gather(t, s)
                scatter(t - 1, (s + 3) % 4)
            return carry

        lax.fori_loop(0, BPT // 4 + 1, body, 0)
        wait_sc(BPT - 1, (BPT - 1) % 4)
        plsc.subcore_barrier()
        pltpu.sync_copy(acc.at[pl.ds(sid * RPS, RPS)],
                        out_h.at[pl.ds(cid * NP + sid * RPS, RPS)])

    return k(table, edge, zeros_rows)


NPK = N // 8
NPPK = NP // 8


def _tc_xw1(x, w1):
    def body(x_ref, w_ref, o_ref):
        o_ref[...] = jnp.dot(x_ref[...], w_ref[...],
                             preferred_element_type=jnp.float32)

    return pl.pallas_call(
        body,
        out_shape=jax.ShapeDtypeStruct((N, 16), jnp.float32),
    )(x, w1)


def _tc_scale(degp, xw1):
    def body(degp_ref, xw_ref, g_ref, dinv_ref):
        deg = degp_ref[pl.ds(0, NPK)] + degp_ref[pl.ds(NPPK, NPK)] + 1.0
        dinv = lax.rsqrt(deg)
        g_ref[...] = xw_ref[...] * dinv
        dinv_ref[...] = dinv

    return pl.pallas_call(
        body,
        out_shape=(jax.ShapeDtypeStruct((NPK, 128), jnp.float32),
                   jax.ShapeDtypeStruct((NPK, 128), jnp.float32)),
    )(degp, xw1)


def _tc_mid(accp, g1, dinv, w2bd, b1p):
    def body(accp_ref, g_ref, dinv_ref, w_ref, b_ref, g2_ref):
        z = dinv_ref[...] * (
            accp_ref[pl.ds(0, NPK)] + accp_ref[pl.ds(NPPK, NPK)] + g_ref[...])
        z = z + b_ref[...]
        h = jnp.maximum(z, 0.0)
        h2 = jnp.dot(h, w_ref[...], preferred_element_type=jnp.float32)
        g2_ref[...] = h2 * dinv_ref[...]

    return pl.pallas_call(
        body,
        out_shape=jax.ShapeDtypeStruct((NPK, 128), jnp.float32),
    )(accp, g1, dinv, w2bd, b1p)


def _tc_final(accn, g2n, dinvn, b2r):
    def body(acc_ref, g_ref, dinv_ref, b_ref, o_ref):
        z = dinv_ref[...] * (
            acc_ref[pl.ds(0, N)] + acc_ref[pl.ds(NP, N)] + g_ref[...])
        z = z + b_ref[...]
        col = lax.broadcasted_iota(jnp.int32, (N, 16), 1)
        zm = jnp.where(col < D_OUT, z, -jnp.inf)
        m = jnp.max(zm, axis=1, keepdims=True)
        e = jnp.exp(zm - m)
        lse = jnp.log(jnp.sum(e, axis=1, keepdims=True)) + m
        o_ref[...] = (z - lse)[:, :D_OUT]

    return pl.pallas_call(
        body,
        out_shape=jax.ShapeDtypeStruct((N, D_OUT), jnp.float32),
    )(accn, g2n, dinvn, b2r)


@jax.jit
def kernel(x, edge_index, W1, b1, W2, b2):
    edge = edge_index.astype(jnp.int32)
    zeros_rows = jnp.zeros((RPS, 16), jnp.float32)
    ones_rows = jnp.ones((EB, 16), jnp.float32)

    xw1 = _tc_xw1(x, W1).reshape(NPK, 128)
    degp = _deg_scatter(edge, ones_rows, zeros_rows)
    g1, dinv = _tc_scale(degp.reshape(2 * NPPK, 128), xw1)

    acc1 = _edge_scatter(g1.reshape(N, 16), edge, zeros_rows)

    w2p = jnp.pad(W2, ((0, 0), (0, 16 - D_OUT)))
    w2bd = jnp.kron(jnp.eye(8, dtype=jnp.float32), w2p)
    b1p = jnp.tile(b1, 8).reshape(1, 128)
    b2r = jnp.pad(b2, (0, 16 - D_OUT)).reshape(1, 16)

    g2 = _tc_mid(acc1.reshape(2 * NPPK, 128), g1, dinv, w2bd, b1p)
    acc2 = _edge_scatter(g2.reshape(N, 16), edge, zeros_rows)
    return _tc_final(acc2.reshape(2 * NP, 16), g2.reshape(N, 16),
                     dinv.reshape(N, 16), b2r)


# speedup vs baseline: 1.0724x; 1.0659x over previous
import functools

import jax
import jax.numpy as jnp
from jax import lax
from jax.experimental import pallas as pl
from jax.experimental.pallas import tpu as pltpu
from jax.experimental.pallas import tpu_sc as plsc

N = 10000
E = 320000
D_IN = 128
D_HID = 16
D_OUT = 7

NC = 2
NS = 16
NW = NC * NS
EB = 128
NBATCH = E // EB
BPT = -(-NBATCH // NW)
NP = 10240
RPS = NP // NS


def _sc_mesh():
    return plsc.VectorSubcoreMesh(
        core_axis_name="c", subcore_axis_name="s",
        num_cores=NC, num_subcores=NS)


def _deg_scatter(edge, ones_rows, zeros_rows):
    @functools.partial(
        pl.kernel,
        out_type=jax.ShapeDtypeStruct((NC * NP, 16), jnp.float32),
        mesh=_sc_mesh(),
        scratch_types=[
            pltpu.VMEM((4, EB), jnp.int32),
            pltpu.VMEM((EB, 16), jnp.float32),
            pltpu.VMEM_SHARED((NP, 16), jnp.float32),
            pltpu.SemaphoreType.DMA((4,)),
            pltpu.SemaphoreType.DMA((4,)),
        ],
        compiler_params=pltpu.CompilerParams(use_tc_tiling_on_sc=False),
    )
    def k(edge_h, ones_h, zeros_h, out_h, didx, ones_v, acc, sem_i, sem_sc):
        cid = lax.axis_index("c")
        sid = lax.axis_index("s")
        wid = sid * NC + cid
        pltpu.sync_copy(ones_h, ones_v)
        pltpu.sync_copy(zeros_h, acc.at[pl.ds(sid * RPS, RPS)])
        plsc.subcore_barrier()

        def valid(t):
            return jnp.logical_and(t >= 0, (wid + t * NW) < NBATCH)

        def boff(t):
            return (wid + t * NW) * EB

        def idx_copy(t, s):
            return pltpu.make_async_copy(
                edge_h.at[1, pl.ds(boff(t), EB)], didx.at[s], sem_i.at[s])

        def sc_desc(s):
            return pltpu.make_async_copy(
                ones_v, acc.at[didx.at[s]], sem_sc.at[s])

        def issue_idx(t, s):
            @pl.when(valid(t))
            def _():
                idx_copy(t, s).start()

        def wait_sc(t, s):
            @pl.when(valid(t))
            def _():
                sc_desc(s).wait()

        def scatter(t, s):
            @pl.when(valid(t))
            def _():
                idx_copy(t, s).wait()
                pltpu.async_copy(ones_v, acc.at[didx.at[s]], sem_sc.at[s],
                                 add=True)

        issue_idx(0, 0)
        issue_idx(1, 1)

        def body(T4, carry):
            T = T4 * 4
            for s in range(4):
                t = T + s
                wait_sc(t - 2, (s + 2) % 4)
                issue_idx(t + 2, (s + 2) % 4)
                scatter(t, s)
            return carry

        lax.fori_loop(0, BPT // 4 + 1, body, 0)
        wait_sc(BPT - 1, (BPT - 1) % 4)
        wait_sc(BPT, BPT % 4)
        plsc.subcore_barrier()
        pltpu.sync_copy(acc.at[pl.ds(sid * RPS, RPS)],
                        out_h.at[pl.ds(cid * NP + sid * RPS, RPS)])

    return k(edge, ones_rows, zeros_rows)


def _edge_scatter(table, edge, zeros_rows):
    @functools.partial(
        pl.kernel,
        out_type=jax.ShapeDtypeStruct((NC * NP, 16), jnp.float32),
        mesh=_sc_mesh(),
        scratch_types=[
            pltpu.VMEM((4, EB), jnp.int32),
            pltpu.VMEM((4, EB), jnp.int32),
            pltpu.VMEM((4, EB, 16), jnp.float32),
            pltpu.VMEM_SHARED((NP, 16), jnp.float32),
            pltpu.VMEM_SHARED((N, 16), jnp.float32),
            pltpu.SemaphoreType.DMA((4,)),
            pltpu.SemaphoreType.DMA((4,)),
            pltpu.SemaphoreType.DMA((4,)),
        ],
        compiler_params=pltpu.CompilerParams(use_tc_tiling_on_sc=False),
    )
    def k(table_h, edge_h, zeros_h, out_h,
          sidx, didx, rows, acc, tbl, sem_i, sem_g, sem_sc):
        cid = lax.axis_index("c")
        sid = lax.axis_index("s")
        wid = sid * NC + cid
        pltpu.sync_copy(zeros_h, acc.at[pl.ds(sid * RPS, RPS)])
        @pl.when(sid < 15)
        def _():
            pltpu.sync_copy(table_h.at[pl.ds(sid * 624, 624)],
                            tbl.at[pl.ds(sid * 624, 624)])

        @pl.when(sid == 15)
        def _():
            pltpu.sync_copy(table_h.at[pl.ds(9360, 640)],
                            tbl.at[pl.ds(9360, 640)])

        plsc.subcore_barrier()

        def valid(t):
            return jnp.logical_and(t >= 0, (wid + t * NW) < NBATCH)

        def boff(t):
            return (wid + t * NW) * EB

        def sidx_copy(t, s):
            return pltpu.make_async_copy(
                edge_h.at[0, pl.ds(boff(t), EB)], sidx.at[s], sem_i.at[s])

        def didx_copy(t, s):
            return pltpu.make_async_copy(
                edge_h.at[1, pl.ds(boff(t), EB)], didx.at[s], sem_i.at[s])

        def gather_copy(s):
            return pltpu.make_async_copy(
                tbl.at[sidx.at[s]], rows.at[s], sem_g.at[s])

        def sc_desc(s):
            return pltpu.make_async_copy(
                rows.at[s], acc.at[didx.at[s]], sem_sc.at[s])

        def issue_idx(t, s):
            @pl.when(valid(t))
            def _():
                sidx_copy(t, s).start()
                didx_copy(t, s).start()

        def wait_sc(t, s):
            @pl.when(valid(t))
            def _():
                sc_desc(s).wait()

        def start_gather(t, s):
            @pl.when(valid(t))
            def _():
                sidx_copy(t, s).wait()
                didx_copy(t, s).wait()
                gather_copy(s).start()

        def scatter(t, s):
            @pl.when(valid(t))
            def _():
                gather_copy(s).wait()
                pltpu.async_copy(rows.at[s], acc.at[didx.at[s]],
                                 sem_sc.at[s], add=True)

        issue_idx(0, 0)
        issue_idx(1, 1)

        def body(T4, carry):
            T = T4 * 4
            for s in range(4):
                t = T + s
                wait_sc(t - 2, (s + 2) % 4)
                issue_idx(t + 2, (s + 2) % 4)
                start_gather(t, s)
                scatter(t - 1, (s + 3) % 4)
            return carry

        lax.fori_loop(0, BPT // 4 + 1, body, 0)
        wait_sc(BPT - 1, (BPT - 1) % 4)
        plsc.subcore_barrier()
        pltpu.sync_copy(acc.at[pl.ds(sid * RPS, RPS)],
                        out_h.at[pl.ds(cid * NP + sid * RPS, RPS)])

    return k(table, edge, zeros_rows)


NPK = N // 8
NPPK = NP // 8


def _tc_xw1(x, w1):
    def body(x_ref, w_ref, o_ref):
        o_ref[...] = jnp.dot(x_ref[...], w_ref[...],
                             preferred_element_type=jnp.float32)

    return pl.pallas_call(
        body,
        out_shape=jax.ShapeDtypeStruct((N, 16), jnp.float32),
    )(x, w1)


def _tc_scale(degp, xw1):
    def body(degp_ref, xw_ref, g_ref, dinv_ref):
        deg = degp_ref[pl.ds(0, NPK)] + degp_ref[pl.ds(NPPK, NPK)] + 1.0
        dinv = lax.rsqrt(deg)
        g_ref[...] = xw_ref[...] * dinv
        dinv_ref[...] = dinv

    return pl.pallas_call(
        body,
        out_shape=(jax.ShapeDtypeStruct((NPK, 128), jnp.float32),
                   jax.ShapeDtypeStruct((NPK, 128), jnp.float32)),
    )(degp, xw1)


def _tc_mid(accp, g1, dinv, w2bd, b1p):
    def body(accp_ref, g_ref, dinv_ref, w_ref, b_ref, g2_ref):
        z = dinv_ref[...] * (
            accp_ref[pl.ds(0, NPK)] + accp_ref[pl.ds(NPPK, NPK)] + g_ref[...])
        z = z + b_ref[...]
        h = jnp.maximum(z, 0.0)
        h2 = jnp.dot(h, w_ref[...], preferred_element_type=jnp.float32)
        g2_ref[...] = h2 * dinv_ref[...]

    return pl.pallas_call(
        body,
        out_shape=jax.ShapeDtypeStruct((NPK, 128), jnp.float32),
    )(accp, g1, dinv, w2bd, b1p)


def _tc_zfinal(accp, g2, dinv, b2p):
    def body(accp_ref, g_ref, dinv_ref, b_ref, o_ref):
        zp = dinv_ref[...] * (
            accp_ref[pl.ds(0, NPK)] + accp_ref[pl.ds(NPPK, NPK)] + g_ref[...])
        o_ref[...] = zp + b_ref[...]

    return pl.pallas_call(
        body,
        out_shape=jax.ShapeDtypeStruct((NPK, 128), jnp.float32),
    )(accp, g2, dinv, b2p)


def _tc_softmax(z):
    def body(z_ref, o_ref):
        z = z_ref[...]
        col = lax.broadcasted_iota(jnp.int32, (N, 16), 1)
        zm = jnp.where(col < D_OUT, z, -jnp.inf)
        m = jnp.max(zm, axis=1, keepdims=True)
        e = jnp.exp(zm - m)
        lse = jnp.log(jnp.sum(e, axis=1, keepdims=True)) + m
        o_ref[...] = (z - lse)[:, :D_OUT]

    return pl.pallas_call(
        body,
        out_shape=jax.ShapeDtypeStruct((N, D_OUT), jnp.float32),
    )(z)


@jax.jit
def kernel(x, edge_index, W1, b1, W2, b2):
    edge = edge_index.astype(jnp.int32)
    zeros_rows = jnp.zeros((RPS, 16), jnp.float32)
    ones_rows = jnp.ones((EB, 16), jnp.float32)

    xw1 = _tc_xw1(x, W1).reshape(NPK, 128)
    degp = _deg_scatter(edge, ones_rows, zeros_rows)
    g1, dinv = _tc_scale(degp.reshape(2 * NPPK, 128), xw1)

    acc1 = _edge_scatter(g1.reshape(N, 16), edge, zeros_rows)

    w2p = jnp.pad(W2, ((0, 0), (0, 16 - D_OUT)))
    w2bd = jnp.kron(jnp.eye(8, dtype=jnp.float32), w2p)
    b1p = jnp.tile(b1, 8).reshape(1, 128)
    b2p = jnp.tile(jnp.pad(b2, (0, 16 - D_OUT)), 8).reshape(1, 128)

    g2 = _tc_mid(acc1.reshape(2 * NPPK, 128), g1, dinv, w2bd, b1p)
    acc2 = _edge_scatter(g2.reshape(N, 16), edge, zeros_rows)
    z2 = _tc_zfinal(acc2.reshape(2 * NPPK, 128), g2, dinv, b2p)
    return _tc_softmax(z2.reshape(N, 16))
